# Initial kernel scaffold; baseline (speedup 1.0000x reference)
import jax
import jax.numpy as jnp
from jax.experimental import pallas as pl


def kernel(x, emb, loc_emb, ln_g, ln_b, W1, b1, bn_g, bn_b, Wh, bh):
    raise NotImplementedError("write your pallas kernel here")


# speedup vs baseline: 14.8570x; 14.8570x over previous
import functools

import jax
import jax.numpy as jnp
from jax import lax
from jax.experimental import pallas as pl
from jax.experimental.pallas import tpu as pltpu
from jax.experimental.pallas import tpu_sc as plsc

B = 16384
D = 16
P = 25
EPS = 1e-5

_info = plsc.get_sparse_core_info()
NC, NS, L = _info.num_cores, _info.num_subcores, _info.num_lanes
NW = NC * NS
BPW = B // NW
G = BPW // L


def _prep_body(emb_ref, loc_ref, lng_ref, lnb_ref, w1_ref, u_ref):
    rt = lax.broadcasted_iota(jnp.int32, (256, 16), 0)
    kt = lax.broadcasted_iota(jnp.int32, (256, 16), 1)
    oh_t = (rt % 10 == kt).astype(jnp.float32)
    rp = lax.broadcasted_iota(jnp.int32, (256, 32), 0)
    kp = lax.broadcasted_iota(jnp.int32, (256, 32), 1)
    oh_p = (rp // 10 == kp).astype(jnp.float32)
    e = (jnp.dot(oh_t, emb_ref[...], preferred_element_type=jnp.float32)
         + jnp.dot(oh_p, loc_ref[...], preferred_element_type=jnp.float32))
    mu = jnp.mean(e, axis=1, keepdims=True)
    var = jnp.mean((e - mu) ** 2, axis=1, keepdims=True)
    ln = (e - mu) * lax.rsqrt(var + EPS) * lng_ref[...] + lnb_ref[...]
    u_ref[...] = jnp.dot(ln, w1_ref[...],
                         preferred_element_type=jnp.float32) * (1.0 / P)


def _prep(emb, loc_emb, ln_g, ln_b, W1):
    emb_p = jnp.pad(emb, ((0, 6), (0, 0)))
    loc_p = jnp.pad(loc_emb, ((0, 7), (0, 0)))
    return pl.pallas_call(
        _prep_body,
        out_shape=jax.ShapeDtypeStruct((256, 16), jnp.float32),
    )(emb_p, loc_p, ln_g.reshape(1, 16), ln_b.reshape(1, 16), W1)


def _sc_body(x_hbm, u_hbm, h1_hbm, h2_hbm, x_v, u_v, h1_v, h2_v):
    wid = lax.axis_index("s") * NC + lax.axis_index("c")
    pltpu.sync_copy(x_hbm.at[pl.ds(wid * (BPW * 2 * P), BPW * 2 * P)], x_v)
    pltpu.sync_copy(u_hbm, u_v)
    i50 = lax.iota(jnp.int32, 16) * (2 * P)

    def gbody(g, carry):
        gb = g * (L * 2 * P)
        for br in range(2):
            accs = [jnp.zeros((16,), jnp.float32) for _ in range(D)]
            for p in range(P):
                xi = i50 + (gb + br * P + p)
                xv = plsc.load_gather(x_v, [xi])
                rv = xv * D + (p * 10 * D)
                for dd in range(D):
                    accs[dd] = accs[dd] + plsc.load_gather(u_v, [rv + dd])
            hv = h1_v if br == 0 else h2_v
            for dd in range(D):
                hv[pl.ds(dd * BPW + g * L, L)] = accs[dd]
        return carry

    lax.fori_loop(0, G, gbody, 0)
    pltpu.sync_copy(h1_v, h1_hbm.at[wid])
    pltpu.sync_copy(h2_v, h2_hbm.at[wid])


_sc_gather = functools.partial(
    pl.kernel,
    out_type=(jax.ShapeDtypeStruct((NW, D * BPW), jnp.float32),
              jax.ShapeDtypeStruct((NW, D * BPW), jnp.float32)),
    mesh=plsc.VectorSubcoreMesh(core_axis_name="c", subcore_axis_name="s"),
    compiler_params=pltpu.CompilerParams(needs_layout_passes=False),
    scratch_types=[
        pltpu.VMEM((BPW * 2 * P,), jnp.int32),
        pltpu.VMEM((4096,), jnp.float32),
        pltpu.VMEM((D * BPW,), jnp.float32),
        pltpu.VMEM((D * BPW,), jnp.float32),
    ],
)(_sc_body)


def _fin_body(h1_ref, h2_ref, bng_ref, bnb_ref, wh_ref, bh_ref, o_ref):
    bng = bng_ref[...]
    bnb = bnb_ref[...]

    def norm(h):
        m = jnp.mean(h, axis=(0, 2), keepdims=True)
        v = jnp.mean((h - m) ** 2, axis=(0, 2), keepdims=True)
        return (h - m) * lax.rsqrt(v + EPS) * bng + bnb

    dh = (norm(h1_ref[...]) - norm(h2_ref[...])) * 0.5
    y = jnp.sum(dh * wh_ref[...], axis=1) + bh_ref[...]
    o_ref[...] = jax.nn.sigmoid(y)


def _finish(h1, h2, bn_g, bn_b, Wh, bh):
    return pl.pallas_call(
        _fin_body,
        out_shape=jax.ShapeDtypeStruct((NW, BPW), jnp.float32),
    )(h1.reshape(NW, D, BPW), h2.reshape(NW, D, BPW),
      bn_g.reshape(1, D, 1), bn_b.reshape(1, D, 1),
      Wh.reshape(1, D, 1), bh.reshape(1, 1))


def kernel(x, emb, loc_emb, ln_g, ln_b, W1, b1, bn_g, bn_b, Wh, bh):
    del b1
    u = _prep(emb, loc_emb, ln_g, ln_b, W1)
    h1, h2 = _sc_gather(x.reshape(B * 2 * P), u.reshape(4096))
    out = _finish(h1, h2, bn_g, bn_b, Wh, bh)
    return out.reshape(B, 1)


# speedup vs baseline: 43.4180x; 2.9224x over previous
import functools

import jax
import jax.numpy as jnp
import numpy as np
from jax import lax
from jax.experimental import pallas as pl
from jax.experimental.pallas import tpu as pltpu
from jax.experimental.pallas import tpu_sc as plsc

B = 16384
D = 16
P = 25
EPS = 1e-5
NQ = 12
NS_ = 13
RT = 1280
SLOTS = 32

_info = plsc.get_sparse_core_info()
NC, NCS, L = _info.num_cores, _info.num_subcores, _info.num_lanes
NW = NC * NCS
BPW = B // NW
G = BPW // L


def _pair_const():
    m = np.zeros((50, SLOTS), np.float32)
    offs = np.zeros((SLOTS,), np.float32)
    for br in range(2):
        for q in range(NQ):
            m[br * P + 2 * q, br * NS_ + q] = 10.0
            m[br * P + 2 * q + 1, br * NS_ + q] = 1.0
            offs[br * NS_ + q] = 100.0 * q
        m[br * P + 24, br * NS_ + NQ] = 1.0
        offs[br * NS_ + NQ] = 1200.0
    return m, offs.reshape(1, SLOTS)


_PM, _POFF = _pair_const()
_NBLK = 16
_BS = B // _NBLK


def _pair_body(x_ref, m_ref, off_ref, o_ref):
    xf = x_ref[...].astype(jnp.float32)
    pr = jnp.dot(xf, m_ref[...], preferred_element_type=jnp.float32)
    pr = pr + off_ref[...]
    v3 = pr.reshape(_BS // 128, 128, SLOTS)
    o_ref[...] = jnp.swapaxes(v3, 1, 2).astype(jnp.int32)


def _pair(x2):
    return pl.pallas_call(
        _pair_body,
        grid=(_NBLK,),
        in_specs=[
            pl.BlockSpec((_BS, 2 * P), lambda i: (i, 0)),
            pl.BlockSpec((2 * P, SLOTS), lambda i: (0, 0)),
            pl.BlockSpec((1, SLOTS), lambda i: (0, 0)),
        ],
        out_specs=pl.BlockSpec((_BS // 128, SLOTS, 128), lambda i: (i, 0, 0)),
        out_shape=jax.ShapeDtypeStruct((B // 128, SLOTS, 128), jnp.int32),
    )(x2, jnp.asarray(_PM), jnp.asarray(_POFF))


def _pair_expand_const() -> np.ndarray:
    j = np.arange(RT)
    q, a, b = j // 100, (j % 100) // 10, j % 10
    col1 = np.where(j < 1200, 20 * q + a, np.where(j < 1210, 240 + (j - 1200), -1))
    col2 = np.where(j < 1200, 20 * q + 10 + b, -1)
    c = np.arange(256)[:, None]
    return ((c == col1[None, :]).astype(np.float32)
            + (c == col2[None, :]).astype(np.float32))


_AT = _pair_expand_const()


def _prep_body(emb_ref, loc_ref, lng_ref, lnb_ref, w1_ref, at_ref, p_ref):
    ct = lax.broadcasted_iota(jnp.int32, (16, 256), 0)
    rt = lax.broadcasted_iota(jnp.int32, (16, 256), 1)
    oh_t = (ct == rt % 10).astype(jnp.float32)
    cp = lax.broadcasted_iota(jnp.int32, (32, 256), 0)
    rp = lax.broadcasted_iota(jnp.int32, (32, 256), 1)
    oh_p = (cp == rp // 10).astype(jnp.float32)
    e = (jnp.dot(emb_ref[...], oh_t, preferred_element_type=jnp.float32)
         + jnp.dot(loc_ref[...], oh_p, preferred_element_type=jnp.float32))
    mu = jnp.mean(e, axis=0, keepdims=True)
    var = jnp.mean((e - mu) ** 2, axis=0, keepdims=True)
    ln = (e - mu) * lax.rsqrt(var + EPS) * lng_ref[...] + lnb_ref[...]
    v = lax.dot_general(w1_ref[...], ln, (((0,), (0,)), ((), ()))) * (1.0 / P)
    p_ref[...] = jnp.dot(v, at_ref[...], preferred_element_type=jnp.float32)


def _prep(emb, loc_emb, ln_g, ln_b, W1):
    emb_t = jnp.pad(emb.T, ((0, 0), (0, 6)))
    loc_t = jnp.pad(loc_emb.T, ((0, 0), (0, 7)))
    return pl.pallas_call(
        _prep_body,
        out_shape=jax.ShapeDtypeStruct((D, RT), jnp.float32),
    )(emb_t, loc_t, ln_g.reshape(D, 1), ln_b.reshape(D, 1), W1,
      jnp.asarray(_AT))


def _sc_body(xp_hbm, u_hbm, h1_hbm, h2_hbm, xp_v, u_v, h1_v, h2_v):
    wid = lax.axis_index("s") * NC + lax.axis_index("c")
    pltpu.sync_copy(xp_hbm.at[pl.ds(wid * (BPW * SLOTS), BPW * SLOTS)], xp_v)
    pltpu.sync_copy(u_hbm, u_v)

    def gbody(g, carry):
        m_off = (g // 8) * (SLOTS * 128) + (g % 8) * L
        for br in range(2):
            accs = [jnp.zeros((16,), jnp.float32) for _ in range(D)]
            for j in range(NS_):
                slot = br * NS_ + j
                xv = xp_v[pl.ds(m_off + slot * 128, L)]
                for dd in range(D):
                    accs[dd] = accs[dd] + plsc.load_gather(u_v, [xv + dd * RT])
            hv = h1_v if br == 0 else h2_v
            for dd in range(D):
                hv[pl.ds(dd * BPW + g * L, L)] = accs[dd]
        return carry

    lax.fori_loop(0, G, gbody, 0)
    for dd in range(D):
        pltpu.sync_copy(h1_v.at[pl.ds(dd * BPW, BPW)], h1_hbm.at[wid * D + dd])
        pltpu.sync_copy(h2_v.at[pl.ds(dd * BPW, BPW)], h2_hbm.at[wid * D + dd])


_sc_gather = functools.partial(
    pl.kernel,
    out_type=(jax.ShapeDtypeStruct((NW * D, BPW), jnp.float32),
              jax.ShapeDtypeStruct((NW * D, BPW), jnp.float32)),
    mesh=plsc.VectorSubcoreMesh(core_axis_name="c", subcore_axis_name="s"),
    compiler_params=pltpu.CompilerParams(needs_layout_passes=False),
    scratch_types=[
        pltpu.VMEM((BPW * SLOTS,), jnp.int32),
        pltpu.VMEM((D * RT,), jnp.float32),
        pltpu.VMEM((D * BPW,), jnp.float32),
        pltpu.VMEM((D * BPW,), jnp.float32),
    ],
)(_sc_body)


_EC = (np.arange(NW * D)[None, :] % D == np.arange(D)[:, None]).astype(np.float32)
_EG = (np.arange(NW * D)[None, :] // D == np.arange(NW)[:, None]).astype(np.float32)


def _fin_body(h1_ref, h2_ref, bng_ref, wh_ref, bh_ref, ec_ref, eg_ref, o_ref):
    ec = ec_ref[...]
    h1 = h1_ref[...]
    h2 = h2_ref[...]

    def coeffs(h):
        s = jnp.dot(ec, h, preferred_element_type=jnp.float32)
        ss = jnp.dot(ec, h * h, preferred_element_type=jnp.float32)
        m = jnp.sum(s, axis=1, keepdims=True) * (1.0 / B)
        v = jnp.sum(ss, axis=1, keepdims=True) * (1.0 / B) - m * m
        a = lax.rsqrt(v + EPS) * bng_ref[...]
        arow = lax.dot_general(ec, a, (((0,), (0,)), ((), ())))
        mrow = lax.dot_general(ec, m, (((0,), (0,)), ((), ())))
        return arow, mrow

    a1, m1 = coeffs(h1)
    a2, m2 = coeffs(h2)
    whrow = lax.dot_general(ec, wh_ref[...], (((0,), (0,)), ((), ())))
    z = ((h1 - m1) * a1 - (h2 - m2) * a2) * (0.5 * whrow)
    y = jnp.dot(eg_ref[...], z, preferred_element_type=jnp.float32) + bh_ref[...]
    o_ref[...] = jax.nn.sigmoid(y)


def _finish(h1, h2, bn_g, Wh, bh):
    return pl.pallas_call(
        _fin_body,
        out_shape=jax.ShapeDtypeStruct((NW, BPW), jnp.float32),
    )(h1, h2, bn_g.reshape(D, 1), Wh.reshape(D, 1), bh.reshape(1, 1),
      jnp.asarray(_EC), jnp.asarray(_EG))


def kernel(x, emb, loc_emb, ln_g, ln_b, W1, b1, bn_g, bn_b, Wh, bh):
    del b1, bn_b
    xp = _pair(x.reshape(B, 2 * P))
    u = _prep(emb, loc_emb, ln_g, ln_b, W1)
    h1, h2 = _sc_gather(xp.reshape(B * SLOTS), u.reshape(D * RT))
    out = _finish(h1, h2, bn_g, Wh, bh)
    return out.reshape(B, 1)
